# Initial kernel scaffold; baseline (speedup 1.0000x reference)
import functools

import jax
import jax.numpy as jnp
import numpy as np
from jax import lax
from jax.experimental import pallas as pl
from jax.experimental.pallas import tpu as pltpu
from jax.experimental.pallas import tpu_sc as plsc

N = 10000
E = 320000
F_IN = 128
H1, C1 = 8, 8
OUT = 128

D1 = H1 * C1
W1ROW = D1 + 16
D2 = OUT
W2ROW = D2 + 16

NC, NS = 2, 16
NW = NC * NS
EPW = E // NW
CHUNK = 80
NCHUNK = EPW // CHUNK
ROWS_PT = N // NS


def _dense_kernel(x_ref, wa_ref, wb_ref, ext_ref, ad_ref):
    x = x_ref[...]
    ext_ref[...] = jnp.dot(x, wa_ref[...], preferred_element_type=jnp.float32)
    ad_ref[...] = jnp.dot(x, wb_ref[...], preferred_element_type=jnp.float32)


def _dense_call(x, wa, wb, block_rows):
    n = x.shape[0]
    return pl.pallas_call(
        _dense_kernel,
        grid=(n // block_rows,),
        in_specs=[
            pl.BlockSpec((block_rows, x.shape[1]), lambda i: (i, 0)),
            pl.BlockSpec(wa.shape, lambda i: (0, 0)),
            pl.BlockSpec(wb.shape, lambda i: (0, 0)),
        ],
        out_specs=[
            pl.BlockSpec((block_rows, wa.shape[1]), lambda i: (i, 0)),
            pl.BlockSpec((block_rows, wb.shape[1]), lambda i: (i, 0)),
        ],
        out_shape=[
            jax.ShapeDtypeStruct((n, wa.shape[1]), jnp.float32),
            jax.ShapeDtypeStruct((n, wb.shape[1]), jnp.float32),
        ],
    )(x, wa, wb)


def _make_edge_kernel(row_w, nfeat16):
    mesh = plsc.VectorSubcoreMesh(
        core_axis_name="c", subcore_axis_name="s", num_cores=NC, num_subcores=NS
    )

    @functools.partial(
        pl.kernel,
        out_type=jax.ShapeDtypeStruct((NC, N, row_w), jnp.float32),
        mesh=mesh,
        scratch_types=[
            pltpu.VMEM_SHARED((N, row_w), jnp.float32),
            pltpu.VMEM((CHUNK,), jnp.int32),
            pltpu.VMEM((CHUNK,), jnp.int32),
            pltpu.VMEM((CHUNK, row_w), jnp.float32),
            pltpu.VMEM((CHUNK, 16), jnp.float32),
            pltpu.VMEM((CHUNK, row_w), jnp.float32),
            pltpu.SemaphoreType.DMA,
        ],
    )
    def edge_kernel(ext_hbm, ad_hbm, src_hbm, dst_hbm, zer_hbm, out_hbm,
                    acc_sh, idx_s, idx_d, extb, adb, outb, sem):
        cid = lax.axis_index("c")
        sid = lax.axis_index("s")
        wid = sid * NC + cid

        pltpu.sync_copy(zer_hbm, acc_sh.at[pl.ds(sid * ROWS_PT, ROWS_PT)])
        plsc.subcore_barrier()

        base = wid * EPW
        fdim = 16 * nfeat16

        def chunk_body(g, carry):
            off = pl.multiple_of(base + g * CHUNK, 8)
            pltpu.sync_copy(src_hbm.at[pl.ds(off, CHUNK)], idx_s)
            pltpu.sync_copy(dst_hbm.at[pl.ds(off, CHUNK)], idx_d)
            pltpu.async_copy(ext_hbm.at[idx_s], extb, sem).wait()
            pltpu.async_copy(ad_hbm.at[idx_d], adb, sem).wait()

            def edge_body(i, c2):
                as16 = extb[i, pl.ds(fdim, 16)]
                ad16 = adb[i, :]
                a = as16 + ad16
                e = jnp.exp(jnp.where(a > 0, a, 0.2 * a))
                outb[i, pl.ds(fdim, 16)] = e
                for j in range(nfeat16):
                    outb[i, pl.ds(16 * j, 16)] = e * extb[i, pl.ds(16 * j, 16)]
                return c2

            lax.fori_loop(0, CHUNK, edge_body, 0)
            pltpu.sync_copy(outb, acc_sh.at[idx_d], add=True)
            return carry

        lax.fori_loop(0, NCHUNK, chunk_body, 0)
        plsc.subcore_barrier()

        pltpu.sync_copy(
            acc_sh.at[pl.ds(sid * ROWS_PT, ROWS_PT)],
            out_hbm.at[cid, pl.ds(sid * ROWS_PT, ROWS_PT)],
        )

    return edge_kernel


_edge1 = _make_edge_kernel(W1ROW, D1 // 16)
_edge2 = _make_edge_kernel(W2ROW, D2 // 16)


def _mid_kernel(acc_ref, msel_ref, b1_ref, w2_ref, s2_ref, d2_ref,
                ext2_ref, ad2_ref):
    a = acc_ref[0] + acc_ref[1]
    denom = jnp.dot(a, msel_ref[...], preferred_element_type=jnp.float32)
    h1 = a[:, :D1] / (denom + 1e-16) + b1_ref[...]
    act = jnp.where(h1 > 0, h1, jnp.exp(jnp.minimum(h1, 0.0)) - 1.0)
    h2 = jnp.dot(act, w2_ref[...], preferred_element_type=jnp.float32)
    as2 = jnp.dot(h2, s2_ref[...], preferred_element_type=jnp.float32)
    ad2 = jnp.dot(h2, d2_ref[...], preferred_element_type=jnp.float32)
    ext2_ref[...] = jnp.concatenate([h2, as2], axis=-1)
    ad2_ref[...] = ad2


def _fin_kernel(acc_ref, msel_ref, b2_ref, out_ref):
    a = acc_ref[0] + acc_ref[1]
    denom = jnp.dot(a, msel_ref[...], preferred_element_type=jnp.float32)
    out_ref[...] = a[:, :D2] / (denom + 1e-16) + b2_ref[...]


def kernel(x, edge_index, W1, att_src1, att_dst1, bias1, W2, att_src2,
           att_dst2, bias2):
    f32 = jnp.float32
    perm1 = np.arange(D1).reshape(H1, C1).T.reshape(-1)
    W1c = W1.reshape(F_IN, H1, C1).transpose(0, 2, 1).reshape(F_IN, D1)
    Ws1 = jnp.einsum("fhc,hc->fh", W1.reshape(F_IN, H1, C1), att_src1)
    Wd1 = jnp.einsum("fhc,hc->fh", W1.reshape(F_IN, H1, C1), att_dst1)
    Wext1 = jnp.concatenate([W1c, Ws1, Ws1], axis=1).astype(f32)
    Wad1 = jnp.concatenate([Wd1, Wd1], axis=1).astype(f32)

    b1c = bias1[perm1].astype(f32)
    W2p = W2[perm1, :].astype(f32)
    S2 = jnp.tile(att_src2.reshape(OUT, 1), (1, 16)).astype(f32)
    Dd2 = jnp.tile(att_dst2.reshape(OUT, 1), (1, 16)).astype(f32)

    msel1 = np.zeros((W1ROW, D1), np.float32)
    for h in range(H1):
        for c in range(C1):
            msel1[D1 + h, c * H1 + h] = 1.0
    msel1 = jnp.asarray(msel1)
    msel2 = np.zeros((W2ROW, D2), np.float32)
    msel2[D2, :] = 1.0
    msel2 = jnp.asarray(msel2)

    src = edge_index[0].astype(jnp.int32)
    dst = edge_index[1].astype(jnp.int32)
    zer1 = jnp.zeros((ROWS_PT, W1ROW), f32)
    zer2 = jnp.zeros((ROWS_PT, W2ROW), f32)

    ext1, ad1 = _dense_call(x.astype(f32), Wext1, Wad1, 1000)
    acc1 = _edge1(ext1, ad1, src, dst, zer1)
    ext2, ad2 = pl.pallas_call(
        _mid_kernel,
        grid=(N // 1000,),
        in_specs=[
            pl.BlockSpec((NC, 1000, W1ROW), lambda i: (0, i, 0)),
            pl.BlockSpec(msel1.shape, lambda i: (0, 0)),
            pl.BlockSpec((D1,), lambda i: (0,)),
            pl.BlockSpec(W2p.shape, lambda i: (0, 0)),
            pl.BlockSpec(S2.shape, lambda i: (0, 0)),
            pl.BlockSpec(Dd2.shape, lambda i: (0, 0)),
        ],
        out_specs=[
            pl.BlockSpec((1000, W2ROW), lambda i: (i, 0)),
            pl.BlockSpec((1000, 16), lambda i: (i, 0)),
        ],
        out_shape=[
            jax.ShapeDtypeStruct((N, W2ROW), f32),
            jax.ShapeDtypeStruct((N, 16), f32),
        ],
    )(acc1, msel1, b1c, W2p, S2, Dd2)
    acc2 = _edge2(ext2, ad2, src, dst, zer2)
    out = pl.pallas_call(
        _fin_kernel,
        grid=(N // 1000,),
        in_specs=[
            pl.BlockSpec((NC, 1000, W2ROW), lambda i: (0, i, 0)),
            pl.BlockSpec(msel2.shape, lambda i: (0, 0)),
            pl.BlockSpec((D2,), lambda i: (0,)),
        ],
        out_specs=pl.BlockSpec((1000, D2), lambda i: (i, 0)),
        out_shape=jax.ShapeDtypeStruct((N, D2), f32),
    )(acc2, msel2, bias2.astype(f32))
    return out


# speedup vs baseline: 54.3226x; 54.3226x over previous
import functools

import jax
import jax.numpy as jnp
import numpy as np
from jax import lax
from jax.experimental import pallas as pl
from jax.experimental.pallas import tpu as pltpu
from jax.experimental.pallas import tpu_sc as plsc

N = 10000
E = 320000
F_IN = 128
H1, C1 = 8, 8
OUT = 128

D1 = H1 * C1
W1ROW = D1 + 16
D2 = OUT

NC, NS = 2, 16
NW = NC * NS
EPW = E // NW
CH1 = 80
CH2 = 80
ROWS_PT = 624
NBLK = ROWS_PT // 16


def _dense_kernel(x_ref, wa_ref, ext_ref):
    ext_ref[...] = jnp.dot(x_ref[...], wa_ref[...],
                           preferred_element_type=jnp.float32)


def _dense_call(x, wa, block_rows):
    n = x.shape[0]
    return pl.pallas_call(
        _dense_kernel,
        grid=(n // block_rows,),
        in_specs=[
            pl.BlockSpec((block_rows, x.shape[1]), lambda i: (i, 0)),
            pl.BlockSpec(wa.shape, lambda i: (0, 0)),
        ],
        out_specs=pl.BlockSpec((block_rows, wa.shape[1]), lambda i: (i, 0)),
        out_shape=jax.ShapeDtypeStruct((n, wa.shape[1]), jnp.float32),
    )(x, wa)


def _sc_mesh():
    return plsc.VectorSubcoreMesh(
        core_axis_name="c", subcore_axis_name="s", num_cores=NC, num_subcores=NS
    )


def _fill_zero(buf, w):
    def zr(r, c):
        for j in range(w // 16):
            buf[r, pl.ds(16 * j, 16)] = jnp.zeros((16,), jnp.float32)
        return c

    lax.fori_loop(0, 16, zr, 0)


def _rows_split(sid, copy_fn):
    r0 = sid * ROWS_PT

    def blk(r, c):
        copy_fn(r0 + 16 * r)
        return c

    lax.fori_loop(0, NBLK, blk, 0)

    @pl.when(sid == NS - 1)
    def _tail():
        copy_fn(NS * ROWS_PT)


def _lane_bcast(vec, lane):
    return vec.at[jnp.full((16,), lane, jnp.int32)].get(
        mode="promise_in_bounds")


@functools.partial(
    pl.kernel,
    out_type=jax.ShapeDtypeStruct((NC, N, W1ROW), jnp.float32),
    mesh=_sc_mesh(),
    compiler_params=pltpu.CompilerParams(needs_layout_passes=False),
    scratch_types=[
        pltpu.VMEM_SHARED((N, W1ROW), jnp.float32),
        pltpu.VMEM((CH1,), jnp.int32),
        pltpu.VMEM((CH1,), jnp.int32),
        pltpu.VMEM((CH1, 128), jnp.float32),
        pltpu.VMEM((CH1, 128), jnp.float32),
        pltpu.VMEM((CH1, W1ROW), jnp.float32),
        pltpu.VMEM((16, W1ROW), jnp.float32),
        pltpu.SemaphoreType.DMA,
    ],
)
def _edge1(tab_hbm, src_hbm, dst_hbm, out_hbm,
           acc_sh, idx_s, idx_d, extb, adb, outb, cpb, sem):
    cid = lax.axis_index("c")
    sid = lax.axis_index("s")
    wid = sid * NC + cid

    _fill_zero(cpb, W1ROW)
    _rows_split(sid, lambda r: pltpu.sync_copy(cpb, acc_sh.at[pl.ds(r, 16)]))
    plsc.subcore_barrier()

    base = wid * EPW

    def chunk_body(g, carry):
        off = pl.multiple_of(base + g * CH1, 8)
        pltpu.sync_copy(src_hbm.at[pl.ds(off, CH1)], idx_s)
        pltpu.sync_copy(dst_hbm.at[pl.ds(off, CH1)], idx_d)
        pltpu.async_copy(tab_hbm.at[idx_s], extb, sem).wait()
        pltpu.async_copy(tab_hbm.at[idx_d], adb, sem).wait()

        def edge_body(i, c2):
            as16 = extb[i, pl.ds(D1, 16)]
            ad16 = adb[i, pl.ds(D1 + 16, 16)]
            a = as16 + ad16
            e = jnp.exp(jnp.where(a > 0, a, 0.2 * a))
            outb[i, pl.ds(D1, 16)] = e
            for j in range(D1 // 16):
                outb[i, pl.ds(16 * j, 16)] = e * extb[i, pl.ds(16 * j, 16)]
            return c2

        lax.fori_loop(0, CH1, edge_body, 0)
        pltpu.sync_copy(outb, acc_sh.at[idx_d], add=True)
        return carry

    lax.fori_loop(0, EPW // CH1, chunk_body, 0)
    plsc.subcore_barrier()

    def _pub(r):
        pltpu.sync_copy(acc_sh.at[pl.ds(r, 16)], cpb)
        pltpu.sync_copy(cpb, out_hbm.at[cid, pl.ds(r, 16)])

    _rows_split(sid, _pub)


@functools.partial(
    pl.kernel,
    out_type=(
        jax.ShapeDtypeStruct((NC, N, D2), jnp.float32),
        jax.ShapeDtypeStruct((NW * N,), jnp.float32),
    ),
    mesh=_sc_mesh(),
    compiler_params=pltpu.CompilerParams(needs_layout_passes=False),
    scratch_types=[
        pltpu.VMEM_SHARED((N, D2), jnp.float32),
        pltpu.VMEM((N,), jnp.float32),
        pltpu.VMEM((N,), jnp.float32),
        pltpu.VMEM((N,), jnp.float32),
        pltpu.VMEM((CH2,), jnp.int32),
        pltpu.VMEM((CH2,), jnp.int32),
        pltpu.VMEM((CH2, D2), jnp.float32),
        pltpu.VMEM((16, D2), jnp.float32),
        pltpu.SemaphoreType.DMA,
    ],
)
def _edge2(h_hbm, asf_hbm, adf_hbm, src_hbm, dst_hbm,
           outm_hbm, outd_hbm, accm_sh, asf, adf, den,
           idx_s, idx_d, hb, cpm, sem):
    cid = lax.axis_index("c")
    sid = lax.axis_index("s")
    wid = sid * NC + cid

    pltpu.sync_copy(asf_hbm, asf)
    pltpu.sync_copy(adf_hbm, adf)

    def _zden(r, c):
        den[pl.ds(16 * r, 16)] = jnp.zeros((16,), jnp.float32)
        return c

    lax.fori_loop(0, N // 16, _zden, 0)
    _fill_zero(cpm, D2)
    _rows_split(sid, lambda r: pltpu.sync_copy(cpm, accm_sh.at[pl.ds(r, 16)]))
    plsc.subcore_barrier()

    base = wid * EPW

    def chunk_body(g, carry):
        off = pl.multiple_of(base + g * CH2, 8)
        pltpu.sync_copy(src_hbm.at[pl.ds(off, CH2)], idx_s)
        pltpu.sync_copy(dst_hbm.at[pl.ds(off, CH2)], idx_d)
        pltpu.async_copy(h_hbm.at[idx_s], hb, sem).wait()

        def group_body(g2, c2):
            idxs16 = idx_s[pl.ds(g2 * 16, 16)]
            idxd16 = idx_d[pl.ds(g2 * 16, 16)]
            va = plsc.load_gather(asf, [idxs16])
            vd = plsc.load_gather(adf, [idxd16])
            a = va + vd
            e16 = jnp.exp(jnp.where(a > 0, a, 0.2 * a))
            plsc.addupdate_scatter(den, [idxd16], e16)
            for i in range(16):
                row = g2 * 16 + i
                ebc = _lane_bcast(e16, i)
                for j in range(D2 // 16):
                    hb[row, pl.ds(16 * j, 16)] = (
                        ebc * hb[row, pl.ds(16 * j, 16)])
            return c2

        lax.fori_loop(0, CH2 // 16, group_body, 0)
        pltpu.sync_copy(hb, accm_sh.at[idx_d], add=True)
        return carry

    lax.fori_loop(0, EPW // CH2, chunk_body, 0)
    plsc.subcore_barrier()

    def _pub(r):
        pltpu.sync_copy(accm_sh.at[pl.ds(r, 16)], cpm)
        pltpu.sync_copy(cpm, outm_hbm.at[cid, pl.ds(r, 16)])

    _rows_split(sid, _pub)
    pltpu.sync_copy(den, outd_hbm.at[pl.ds(wid * N, N)])


def _mid_kernel(acc_ref, msel_ref, b1_ref, w2_ref, sd2_ref,
                h2_ref, asad_ref):
    a = acc_ref[0] + acc_ref[1]
    denom = jnp.dot(a, msel_ref[...], preferred_element_type=jnp.float32)
    h1 = a[:, :D1] / (denom + 1e-16) + b1_ref[...]
    act = jnp.where(h1 > 0, h1, jnp.exp(jnp.minimum(h1, 0.0)) - 1.0)
    h2 = jnp.dot(act, w2_ref[...], preferred_element_type=jnp.float32)
    h2_ref[...] = h2
    asad_ref[...] = jnp.dot(h2, sd2_ref[...],
                            preferred_element_type=jnp.float32)


def _fin_kernel(accm_ref, dd_ref, b2_ref, out_ref):
    a = accm_ref[0] + accm_ref[1]
    d = jnp.sum(dd_ref[...], axis=1)
    out_ref[...] = a / (d[:, None] + 1e-16) + b2_ref[...]


def kernel(x, edge_index, W1, att_src1, att_dst1, bias1, W2, att_src2,
           att_dst2, bias2):
    f32 = jnp.float32
    perm1 = np.arange(D1).reshape(H1, C1).T.reshape(-1)
    W1c = W1.reshape(F_IN, H1, C1).transpose(0, 2, 1).reshape(F_IN, D1)
    Ws1 = jnp.einsum("fhc,hc->fh", W1.reshape(F_IN, H1, C1), att_src1)
    Wd1 = jnp.einsum("fhc,hc->fh", W1.reshape(F_IN, H1, C1), att_dst1)
    Wext1 = jnp.concatenate(
        [W1c, Ws1, Ws1, Wd1, Wd1, jnp.zeros((F_IN, 32), f32)],
        axis=1).astype(f32)

    b1c = bias1[perm1].astype(f32)
    W2p = W2[perm1, :].astype(f32)
    Sd2 = jnp.concatenate(
        [jnp.tile(att_src2.reshape(OUT, 1), (1, 8)),
         jnp.tile(att_dst2.reshape(OUT, 1), (1, 8))], axis=1).astype(f32)

    msel1 = np.zeros((W1ROW, D1), np.float32)
    for h in range(H1):
        for c in range(C1):
            msel1[D1 + h, c * H1 + h] = 1.0
    msel1 = jnp.asarray(msel1)
    src = edge_index[0].astype(jnp.int32)
    dst = edge_index[1].astype(jnp.int32)

    tab1 = _dense_call(x.astype(f32), Wext1, 1000)
    acc1 = _edge1(tab1, src, dst)
    h2, asad = pl.pallas_call(
        _mid_kernel,
        grid=(N // 1000,),
        in_specs=[
            pl.BlockSpec((NC, 1000, W1ROW), lambda i: (0, i, 0)),
            pl.BlockSpec(msel1.shape, lambda i: (0, 0)),
            pl.BlockSpec((D1,), lambda i: (0,)),
            pl.BlockSpec(W2p.shape, lambda i: (0, 0)),
            pl.BlockSpec(Sd2.shape, lambda i: (0, 0)),
        ],
        out_specs=[
            pl.BlockSpec((1000, D2), lambda i: (i, 0)),
            pl.BlockSpec((1000, 16), lambda i: (i, 0)),
        ],
        out_shape=[
            jax.ShapeDtypeStruct((N, D2), f32),
            jax.ShapeDtypeStruct((N, 16), f32),
        ],
    )(acc1, msel1, b1c, W2p, Sd2)
    acc2m, denflat = _edge2(h2, asad[:, 0], asad[:, 8], src, dst)
    out = pl.pallas_call(
        _fin_kernel,
        grid=(N // 1000,),
        in_specs=[
            pl.BlockSpec((NC, 1000, D2), lambda i: (0, i, 0)),
            pl.BlockSpec((1000, NW), lambda i: (i, 0)),
            pl.BlockSpec((D2,), lambda i: (0,)),
        ],
        out_specs=pl.BlockSpec((1000, D2), lambda i: (i, 0)),
        out_shape=jax.ShapeDtypeStruct((N, D2), f32),
    )(acc2m, denflat.reshape(NW, N).T, bias2.astype(f32))
    return out


# speedup vs baseline: 56.5612x; 1.0412x over previous
import functools

import jax
import jax.numpy as jnp
import numpy as np
from jax import lax
from jax.experimental import pallas as pl
from jax.experimental.pallas import tpu as pltpu
from jax.experimental.pallas import tpu_sc as plsc

N = 10000
E = 320000
F_IN = 128
H1, C1 = 8, 8
OUT = 128

D1 = H1 * C1
W1ROW = D1 + 16
D2 = OUT

NC, NS = 2, 16
NW = NC * NS
EPW = E // NW
CH1 = 40
NCH1 = EPW // CH1
CH2 = 64
NCH2 = EPW // CH2
TAIL2 = EPW - NCH2 * CH2
ROWS_PT = 624
NBLK = ROWS_PT // 16


def _dense_kernel(x_ref, wa_ref, ext_ref):
    ext_ref[...] = jnp.dot(x_ref[...], wa_ref[...],
                           preferred_element_type=jnp.float32)


def _dense_call(x, wa, block_rows):
    n = x.shape[0]
    return pl.pallas_call(
        _dense_kernel,
        grid=(n // block_rows,),
        in_specs=[
            pl.BlockSpec((block_rows, x.shape[1]), lambda i: (i, 0)),
            pl.BlockSpec(wa.shape, lambda i: (0, 0)),
        ],
        out_specs=pl.BlockSpec((block_rows, wa.shape[1]), lambda i: (i, 0)),
        out_shape=jax.ShapeDtypeStruct((n, wa.shape[1]), jnp.float32),
    )(x, wa)


def _sc_mesh():
    return plsc.VectorSubcoreMesh(
        core_axis_name="c", subcore_axis_name="s", num_cores=NC, num_subcores=NS
    )


def _fill_zero(buf, w):
    def zr(r, c):
        for j in range(w // 16):
            buf[r, pl.ds(16 * j, 16)] = jnp.zeros((16,), jnp.float32)
        return c

    lax.fori_loop(0, 16, zr, 0)


def _rows_split(sid, copy_fn):
    r0 = sid * ROWS_PT

    def blk(r, c):
        copy_fn(r0 + 16 * r)
        return c

    lax.fori_loop(0, NBLK, blk, 0)

    @pl.when(sid == NS - 1)
    def _tail():
        copy_fn(NS * ROWS_PT)


def _lane_bcast(vec, lane):
    return vec.at[jnp.full((16,), lane, jnp.int32)].get(
        mode="promise_in_bounds")


@functools.partial(
    pl.kernel,
    out_type=jax.ShapeDtypeStruct((NC, N, W1ROW), jnp.float32),
    mesh=_sc_mesh(),
    compiler_params=pltpu.CompilerParams(needs_layout_passes=False),
    scratch_types=[
        pltpu.VMEM_SHARED((N, W1ROW), jnp.float32),
        pltpu.VMEM((CH1,), jnp.int32),
        pltpu.VMEM((CH1,), jnp.int32),
        pltpu.VMEM((CH1,), jnp.int32),
        pltpu.VMEM((CH1, 128), jnp.float32),
        pltpu.VMEM((CH1, 128), jnp.float32),
        pltpu.VMEM((CH1, 128), jnp.float32),
        pltpu.VMEM((CH1, 128), jnp.float32),
        pltpu.VMEM((CH1, W1ROW), jnp.float32),
        pltpu.VMEM((CH1, W1ROW), jnp.float32),
        pltpu.VMEM((16, W1ROW), jnp.float32),
        pltpu.SemaphoreType.DMA,
        pltpu.SemaphoreType.DMA,
    ],
)
def _edge1(tab_hbm, src_hbm, dst_hbm, out_hbm,
           acc_sh, idx_s, idx_d0, idx_d1, extb0, extb1, adb0, adb1,
           outb0, outb1, cpb, sem0, sem1):
    cid = lax.axis_index("c")
    sid = lax.axis_index("s")
    wid = sid * NC + cid

    _fill_zero(cpb, W1ROW)
    _rows_split(sid, lambda r: pltpu.sync_copy(cpb, acc_sh.at[pl.ds(r, 16)]))
    plsc.subcore_barrier()

    base = wid * EPW
    idx_d = (idx_d0, idx_d1)
    extb = (extb0, extb1)
    adb = (adb0, adb1)
    outb = (outb0, outb1)
    sems = (sem0, sem1)

    def fire(g, b):
        off = pl.multiple_of(base + g * CH1, 8)
        pltpu.sync_copy(src_hbm.at[pl.ds(off, CH1)], idx_s)
        pltpu.sync_copy(dst_hbm.at[pl.ds(off, CH1)], idx_d[b])
        pltpu.async_copy(tab_hbm.at[idx_s], extb[b], sems[b])
        pltpu.async_copy(tab_hbm.at[idx_d[b]], adb[b], sems[b])

    fire(0, 0)

    def pair_body(k, carry):
        for b in range(2):
            g = 2 * k + b
            nb = 1 - b
            pltpu.make_async_copy(tab_hbm.at[idx_s], extb[b], sems[b]).wait()
            pltpu.make_async_copy(tab_hbm.at[idx_d[b]], adb[b], sems[b]).wait()

            @pl.when(g + 1 < NCH1)
            def _prefetch():
                fire(g + 1, nb)

            def edge_body(i, c2):
                as16 = extb[b][i, pl.ds(D1, 16)]
                ad16 = adb[b][i, pl.ds(D1 + 16, 16)]
                a = as16 + ad16
                e = jnp.exp(jnp.where(a > 0, a, 0.2 * a))
                outb[b][i, pl.ds(D1, 16)] = e
                for j in range(D1 // 16):
                    outb[b][i, pl.ds(16 * j, 16)] = (
                        e * extb[b][i, pl.ds(16 * j, 16)])
                return c2

            lax.fori_loop(0, CH1, edge_body, 0)
            pltpu.sync_copy(outb[b], acc_sh.at[idx_d[b]], add=True)
        return carry

    lax.fori_loop(0, NCH1 // 2, pair_body, 0)
    plsc.subcore_barrier()

    def _pub(r):
        pltpu.sync_copy(acc_sh.at[pl.ds(r, 16)], cpb)
        pltpu.sync_copy(cpb, out_hbm.at[cid, pl.ds(r, 16)])

    _rows_split(sid, _pub)


@functools.partial(
    pl.kernel,
    out_type=(
        jax.ShapeDtypeStruct((NC, N, D2), jnp.float32),
        jax.ShapeDtypeStruct((NW * N,), jnp.float32),
    ),
    mesh=_sc_mesh(),
    compiler_params=pltpu.CompilerParams(needs_layout_passes=False),
    scratch_types=[
        pltpu.VMEM_SHARED((N, D2), jnp.float32),
        pltpu.VMEM((N,), jnp.float32),
        pltpu.VMEM((N,), jnp.float32),
        pltpu.VMEM((N,), jnp.float32),
        pltpu.VMEM((CH2,), jnp.int32),
        pltpu.VMEM((CH2,), jnp.int32),
        pltpu.VMEM((CH2,), jnp.int32),
        pltpu.VMEM((CH2,), jnp.int32),
        pltpu.VMEM((TAIL2,), jnp.int32),
        pltpu.VMEM((TAIL2,), jnp.int32),
        pltpu.VMEM((CH2, D2), jnp.float32),
        pltpu.VMEM((CH2, D2), jnp.float32),
        pltpu.VMEM((16, D2), jnp.float32),
        pltpu.SemaphoreType.DMA,
        pltpu.SemaphoreType.DMA,
    ],
)
def _edge2(h_hbm, asf_hbm, adf_hbm, src_hbm, dst_hbm,
           outm_hbm, outd_hbm, accm_sh, asf, adf, den,
           idx_s0, idx_s1, idx_d0, idx_d1, idx_st, idx_dt,
           hb0, hb1, cpm, sem0, sem1):
    cid = lax.axis_index("c")
    sid = lax.axis_index("s")
    wid = sid * NC + cid

    pltpu.sync_copy(asf_hbm, asf)
    pltpu.sync_copy(adf_hbm, adf)

    def _zden(r, c):
        den[pl.ds(16 * r, 16)] = jnp.zeros((16,), jnp.float32)
        return c

    lax.fori_loop(0, N // 16, _zden, 0)
    _fill_zero(cpm, D2)
    _rows_split(sid, lambda r: pltpu.sync_copy(cpm, accm_sh.at[pl.ds(r, 16)]))
    plsc.subcore_barrier()

    base = wid * EPW
    idx_s = (idx_s0, idx_s1)
    idx_d = (idx_d0, idx_d1)
    hb = (hb0, hb1)
    sems = (sem0, sem1)

    def fire(g, b):
        off = pl.multiple_of(base + g * CH2, 8)
        pltpu.sync_copy(src_hbm.at[pl.ds(off, CH2)], idx_s[b])
        pltpu.sync_copy(dst_hbm.at[pl.ds(off, CH2)], idx_d[b])
        pltpu.async_copy(h_hbm.at[idx_s[b]], hb[b], sems[b])

    def compute(buf, isv, idv, ngroups):
        def group_body(g2, c2):
            idxs16 = isv[pl.ds(g2 * 16, 16)]
            idxd16 = idv[pl.ds(g2 * 16, 16)]
            va = plsc.load_gather(asf, [idxs16])
            vd = plsc.load_gather(adf, [idxd16])
            a = va + vd
            e16 = jnp.exp(jnp.where(a > 0, a, 0.2 * a))
            plsc.addupdate_scatter(den, [idxd16], e16)
            for i in range(16):
                row = g2 * 16 + i
                ebc = _lane_bcast(e16, i)
                for j in range(D2 // 16):
                    buf[row, pl.ds(16 * j, 16)] = (
                        ebc * buf[row, pl.ds(16 * j, 16)])
            return c2

        lax.fori_loop(0, ngroups, group_body, 0)

    fire(0, 0)

    def pair_body(k, carry):
        for b in range(2):
            g = 2 * k + b
            nb = 1 - b
            pltpu.make_async_copy(h_hbm.at[idx_s[b]], hb[b], sems[b]).wait()

            @pl.when(g + 1 < NCH2)
            def _prefetch():
                fire(g + 1, nb)

            compute(hb[b], idx_s[b], idx_d[b], CH2 // 16)
            pltpu.sync_copy(hb[b], accm_sh.at[idx_d[b]], add=True)
        return carry

    lax.fori_loop(0, NCH2 // 2, pair_body, 0)

    offt = pl.multiple_of(base + NCH2 * CH2, 8)
    pltpu.sync_copy(src_hbm.at[pl.ds(offt, TAIL2)], idx_st)
    pltpu.sync_copy(dst_hbm.at[pl.ds(offt, TAIL2)], idx_dt)
    pltpu.async_copy(h_hbm.at[idx_st], hb0.at[pl.ds(0, TAIL2)], sem0).wait()
    compute(hb0, idx_st, idx_dt, TAIL2 // 16)
    pltpu.sync_copy(hb0.at[pl.ds(0, TAIL2)], accm_sh.at[idx_dt], add=True)

    plsc.subcore_barrier()

    def _pub(r):
        pltpu.sync_copy(accm_sh.at[pl.ds(r, 16)], cpm)
        pltpu.sync_copy(cpm, outm_hbm.at[cid, pl.ds(r, 16)])

    _rows_split(sid, _pub)
    pltpu.sync_copy(den, outd_hbm.at[pl.ds(wid * N, N)])


def _mid_kernel(acc_ref, msel_ref, b1_ref, w2_ref, sd2_ref,
                h2_ref, asad_ref):
    a = acc_ref[0] + acc_ref[1]
    denom = jnp.dot(a, msel_ref[...], preferred_element_type=jnp.float32)
    h1 = a[:, :D1] / (denom + 1e-16) + b1_ref[...]
    act = jnp.where(h1 > 0, h1, jnp.exp(jnp.minimum(h1, 0.0)) - 1.0)
    h2 = jnp.dot(act, w2_ref[...], preferred_element_type=jnp.float32)
    h2_ref[...] = h2
    asad_ref[...] = jnp.dot(h2, sd2_ref[...],
                            preferred_element_type=jnp.float32)


def _fin_kernel(accm_ref, dd_ref, b2_ref, out_ref):
    a = accm_ref[0] + accm_ref[1]
    d = jnp.sum(dd_ref[...], axis=1)
    out_ref[...] = a / (d[:, None] + 1e-16) + b2_ref[...]


def kernel(x, edge_index, W1, att_src1, att_dst1, bias1, W2, att_src2,
           att_dst2, bias2):
    f32 = jnp.float32
    perm1 = np.arange(D1).reshape(H1, C1).T.reshape(-1)
    W1c = W1.reshape(F_IN, H1, C1).transpose(0, 2, 1).reshape(F_IN, D1)
    Ws1 = jnp.einsum("fhc,hc->fh", W1.reshape(F_IN, H1, C1), att_src1)
    Wd1 = jnp.einsum("fhc,hc->fh", W1.reshape(F_IN, H1, C1), att_dst1)
    Wext1 = jnp.concatenate(
        [W1c, Ws1, Ws1, Wd1, Wd1, jnp.zeros((F_IN, 32), f32)],
        axis=1).astype(f32)

    b1c = bias1[perm1].astype(f32)
    W2p = W2[perm1, :].astype(f32)
    Sd2 = jnp.concatenate(
        [jnp.tile(att_src2.reshape(OUT, 1), (1, 8)),
         jnp.tile(att_dst2.reshape(OUT, 1), (1, 8))], axis=1).astype(f32)

    msel1 = np.zeros((W1ROW, D1), np.float32)
    for h in range(H1):
        for c in range(C1):
            msel1[D1 + h, c * H1 + h] = 1.0
    msel1 = jnp.asarray(msel1)
    src = edge_index[0].astype(jnp.int32)
    dst = edge_index[1].astype(jnp.int32)

    tab1 = _dense_call(x.astype(f32), Wext1, 1000)
    acc1 = _edge1(tab1, src, dst)
    h2, asad = pl.pallas_call(
        _mid_kernel,
        grid=(N // 1000,),
        in_specs=[
            pl.BlockSpec((NC, 1000, W1ROW), lambda i: (0, i, 0)),
            pl.BlockSpec(msel1.shape, lambda i: (0, 0)),
            pl.BlockSpec((D1,), lambda i: (0,)),
            pl.BlockSpec(W2p.shape, lambda i: (0, 0)),
            pl.BlockSpec(Sd2.shape, lambda i: (0, 0)),
        ],
        out_specs=[
            pl.BlockSpec((1000, D2), lambda i: (i, 0)),
            pl.BlockSpec((1000, 16), lambda i: (i, 0)),
        ],
        out_shape=[
            jax.ShapeDtypeStruct((N, D2), f32),
            jax.ShapeDtypeStruct((N, 16), f32),
        ],
    )(acc1, msel1, b1c, W2p, Sd2)
    acc2m, denflat = _edge2(h2, asad[:, 0], asad[:, 8], src, dst)
    out = pl.pallas_call(
        _fin_kernel,
        grid=(N // 1000,),
        in_specs=[
            pl.BlockSpec((NC, 1000, D2), lambda i: (0, i, 0)),
            pl.BlockSpec((1000, NW), lambda i: (i, 0)),
            pl.BlockSpec((D2,), lambda i: (0,)),
        ],
        out_specs=pl.BlockSpec((1000, D2), lambda i: (i, 0)),
        out_shape=jax.ShapeDtypeStruct((N, D2), f32),
    )(acc2m, denflat.reshape(NW, N).T, bias2.astype(f32))
    return out
